# baseline (speedup 1.0000x reference)
import jax
import jax.numpy as jnp
from jax.experimental import pallas as pl
from jax.experimental.pallas import tpu as pltpu

_N = 32768
_B = 16
_D = 16
_T = 4096
_TS = _T
_NT = _N // _T
_OUT = 256


def _fused_kernel(bounds_ref, seg_ref, xa_ref, W1_ref, W2_ref, W3_ref,
                  Wh_ref, bh_ref, y_ref, acc_ref, cnt_ref):
    i = pl.program_id(0)

    @pl.when(i == 0)
    def _():
        acc_ref[...] = jnp.zeros_like(acc_ref)
        cnt_ref[...] = jnp.zeros_like(cnt_ref)

    for s in range(1):
        ones = jnp.ones((1, _TS), jnp.bfloat16)
        xa = xa_ref[:, s * _TS:(s + 1) * _TS]
        h = jnp.maximum(
            jnp.dot(W1_ref[...], xa, preferred_element_type=jnp.float32),
            0.0).astype(jnp.bfloat16)
        h = jnp.concatenate([h, ones], axis=0)
        h = jnp.maximum(
            jnp.dot(W2_ref[...], h, preferred_element_type=jnp.float32),
            0.0).astype(jnp.bfloat16)
        h = jnp.concatenate([h, ones], axis=0)
        ft = jnp.dot(W3_ref[...], h,
                     preferred_element_type=jnp.float32).astype(jnp.bfloat16)
        f = jnp.transpose(ft, (1, 0))

        seg = seg_ref[0][:, s * _TS:(s + 1) * _TS]
        bidx = jax.lax.broadcasted_iota(jnp.int32, (_B, 1), 0)
        mt = (seg == bidx).astype(jnp.bfloat16)
        cnt_ref[...] += jnp.sum(mt.astype(jnp.float32), axis=1, keepdims=True)

        lo = bounds_ref[i, 2 * s]
        hi = bounds_ref[i, 2 * s + 1]
        mid = jnp.minimum(lo + 1, _B - 1)
        rare = (hi - lo) >= 3

        m_lo = (seg == lo).astype(jnp.bfloat16)
        m_mid = (seg == mid).astype(jnp.bfloat16)
        lhs = jnp.concatenate([ft, ft * m_lo, ft * m_mid], axis=0)
        g_cat = jax.lax.dot_general(lhs, f, (((1,), (0,)), ((), ())),
                                    preferred_element_type=jnp.float32)
        g_full = g_cat[:_D, :]
        g_lo = g_cat[_D:2 * _D, :]
        g_mid = g_cat[2 * _D:, :]
        zero = jnp.where(rare, 0.0, 1.0)
        acc_ref[pl.ds(lo * _D, _D), :] += g_lo * zero
        acc_ref[pl.ds(mid * _D, _D), :] += g_mid * zero
        acc_ref[pl.ds(hi * _D, _D), :] += (g_full - g_lo - g_mid) * zero

        @pl.when(rare)
        def _(ft=ft, f=f, mt=mt):
            lhs_all = jnp.concatenate(
                [ft * mt[b:b + 1, :] for b in range(_B)], axis=0)
            acc_ref[...] += jax.lax.dot_general(
                lhs_all, f, (((1,), (0,)), ((), ())),
                preferred_element_type=jnp.float32)

    @pl.when(i == _NT - 1)
    def _():
        maxc = jnp.max(cnt_ref[...])
        cov = acc_ref[...] / maxc
        p = jnp.sign(cov) * jnp.sqrt(jnp.abs(cov) + 1e-12)
        pflat = jnp.concatenate(
            [jnp.concatenate([p[_D * b + k:_D * b + k + 1, :]
                              for k in range(_D)], axis=1)
             for b in range(_B)], axis=0)
        y = jnp.dot(pflat, Wh_ref[...], preferred_element_type=jnp.float32) + bh_ref[...]
        nrm = jnp.sqrt(jnp.sum(y * y, axis=1, keepdims=True))
        y_ref[...] = y / (nrm + 1e-12)


def kernel(points, segment_ids, W1, b1, W2, b2, W3, b3, Wh, bh):
    seg = segment_ids.astype(jnp.int32)
    bounds = jnp.stack([seg[::_TS], seg[_TS - 1::_TS]], axis=1)
    xa = jnp.concatenate(
        [points.T, jnp.ones((1, _N), jnp.float32)],
        axis=0).astype(jnp.bfloat16)
    W1a = jnp.concatenate([W1, b1[None, :]], axis=0).T.astype(jnp.bfloat16)
    W2a = jnp.concatenate([W2, b2[None, :]], axis=0).T.astype(jnp.bfloat16)
    W3a = jnp.concatenate([W3, b3[None, :]], axis=0).T.astype(jnp.bfloat16)
    out = pl.pallas_call(
        _fused_kernel,
        grid=(_NT,),
        in_specs=[
            pl.BlockSpec(memory_space=pltpu.SMEM),
            pl.BlockSpec((1, 1, _T), lambda i: (i, 0, 0)),
            pl.BlockSpec((5, _T), lambda i: (0, i)),
            pl.BlockSpec((64, 5), lambda i: (0, 0)),
            pl.BlockSpec((64, 65), lambda i: (0, 0)),
            pl.BlockSpec((_D, 65), lambda i: (0, 0)),
            pl.BlockSpec((_D * _D, _OUT), lambda i: (0, 0)),
            pl.BlockSpec((1, _OUT), lambda i: (0, 0)),
        ],
        out_specs=pl.BlockSpec((_B, _OUT), lambda i: (0, 0)),
        out_shape=jax.ShapeDtypeStruct((_B, _OUT), jnp.float32),
        scratch_shapes=[
            pltpu.VMEM((_B * _D, _D), jnp.float32),
            pltpu.VMEM((_B, 1), jnp.float32),
        ],
    )(bounds, seg.reshape(_NT, 1, _T), xa, W1a, W2a, W3a, Wh,
      bh.reshape(1, -1))
    return out


# speedup vs baseline: 1.0219x; 1.0219x over previous
import jax
import jax.numpy as jnp
from jax.experimental import pallas as pl
from jax.experimental.pallas import tpu as pltpu

_N = 32768
_B = 16
_D = 16
_T = 4096
_TS = _T
_NT = _N // _T
_OUT = 256


def _fused_kernel(bounds_ref, seg_ref, xa_ref, W1_ref, W2_ref, W3_ref,
                  Wh_ref, bh_ref, y_ref, acc_ref, cnt_ref):
    i = pl.program_id(0)

    @pl.when(i == 0)
    def _():
        acc_ref[...] = jnp.zeros_like(acc_ref)
        cnt_ref[...] = jnp.zeros_like(cnt_ref)

    for s in range(1):
        ones = jnp.ones((1, _TS), jnp.bfloat16)
        xa = xa_ref[:, s * _TS:(s + 1) * _TS]
        h = jnp.maximum(
            jnp.dot(W1_ref[...], xa, preferred_element_type=jnp.float32),
            0.0).astype(jnp.bfloat16)
        h = jnp.concatenate([h, ones], axis=0)
        h = jnp.maximum(
            jnp.dot(W2_ref[...], h, preferred_element_type=jnp.float32),
            0.0).astype(jnp.bfloat16)
        h = jnp.concatenate([h, ones], axis=0)
        ft = jnp.dot(W3_ref[...], h,
                     preferred_element_type=jnp.float32).astype(jnp.bfloat16)
        f = jnp.transpose(ft, (1, 0))

        seg = seg_ref[0][:, s * _TS:(s + 1) * _TS]
        bidx = jax.lax.broadcasted_iota(jnp.int32, (_B, 1), 0)
        mt = (seg == bidx).astype(jnp.bfloat16)
        cnt_ref[...] += jnp.sum(mt.astype(jnp.float32), axis=1, keepdims=True)

        lo = bounds_ref[i, 2 * s]
        hi = bounds_ref[i, 2 * s + 1]
        mid = jnp.minimum(lo + 1, _B - 1)
        rare = (hi - lo) >= 3

        g_full = jax.lax.dot_general(ft, f, (((1,), (0,)), ((), ())),
                                     preferred_element_type=jnp.float32)
        m_lo = (seg == lo).astype(jnp.bfloat16)
        g_lo = jax.lax.dot_general(ft * m_lo, f, (((1,), (0,)), ((), ())),
                                   preferred_element_type=jnp.float32)
        m_mid = (seg == mid).astype(jnp.bfloat16)
        g_mid = jax.lax.dot_general(ft * m_mid, f, (((1,), (0,)), ((), ())),
                                    preferred_element_type=jnp.float32)
        zero = jnp.where(rare, 0.0, 1.0)
        acc_ref[pl.ds(lo * _D, _D), :] += g_lo * zero
        acc_ref[pl.ds(mid * _D, _D), :] += g_mid * zero
        acc_ref[pl.ds(hi * _D, _D), :] += (g_full - g_lo - g_mid) * zero

        @pl.when(rare)
        def _(ft=ft, f=f, mt=mt):
            lhs_all = jnp.concatenate(
                [ft * mt[b:b + 1, :] for b in range(_B)], axis=0)
            acc_ref[...] += jax.lax.dot_general(
                lhs_all, f, (((1,), (0,)), ((), ())),
                preferred_element_type=jnp.float32)

    @pl.when(i == _NT - 1)
    def _():
        maxc = jnp.max(cnt_ref[...])
        cov = acc_ref[...] / maxc
        p = jnp.sign(cov) * jnp.sqrt(jnp.abs(cov) + 1e-12)
        pflat = jnp.concatenate(
            [jnp.concatenate([p[_D * b + k:_D * b + k + 1, :]
                              for k in range(_D)], axis=1)
             for b in range(_B)], axis=0)
        y = jnp.dot(pflat, Wh_ref[...], preferred_element_type=jnp.float32) + bh_ref[...]
        nrm = jnp.sqrt(jnp.sum(y * y, axis=1, keepdims=True))
        y_ref[...] = y / (nrm + 1e-12)


def kernel(points, segment_ids, W1, b1, W2, b2, W3, b3, Wh, bh):
    seg = segment_ids.astype(jnp.int32)
    bounds = jnp.stack([seg[::_TS], seg[_TS - 1::_TS]], axis=1)
    xa = jnp.concatenate(
        [points.T, jnp.ones((1, _N), jnp.float32)],
        axis=0).astype(jnp.bfloat16)
    W1a = jnp.concatenate([W1, b1[None, :]], axis=0).T.astype(jnp.bfloat16)
    W2a = jnp.concatenate([W2, b2[None, :]], axis=0).T.astype(jnp.bfloat16)
    W3a = jnp.concatenate([W3, b3[None, :]], axis=0).T.astype(jnp.bfloat16)
    out = pl.pallas_call(
        _fused_kernel,
        grid=(_NT,),
        in_specs=[
            pl.BlockSpec(memory_space=pltpu.SMEM),
            pl.BlockSpec((1, 1, _T), lambda i: (i, 0, 0)),
            pl.BlockSpec((5, _T), lambda i: (0, i)),
            pl.BlockSpec((64, 5), lambda i: (0, 0)),
            pl.BlockSpec((64, 65), lambda i: (0, 0)),
            pl.BlockSpec((_D, 65), lambda i: (0, 0)),
            pl.BlockSpec((_D * _D, _OUT), lambda i: (0, 0)),
            pl.BlockSpec((1, _OUT), lambda i: (0, 0)),
        ],
        out_specs=pl.BlockSpec((_B, _OUT), lambda i: (0, 0)),
        out_shape=jax.ShapeDtypeStruct((_B, _OUT), jnp.float32),
        scratch_shapes=[
            pltpu.VMEM((_B * _D, _D), jnp.float32),
            pltpu.VMEM((_B, 1), jnp.float32),
        ],
    )(bounds, seg.reshape(_NT, 1, _T), xa, W1a, W2a, W3a, Wh,
      bh.reshape(1, -1))
    return out
